# baseline (speedup 1.0000x reference)
import jax
import jax.numpy as jnp
from jax.experimental import pallas as pl

_B, _S, _D = 4, 4096, 2048
_N = _B * _S
_IDX = 3
_ROWS = 512


def _tc_body(in_ref, out_ref):
    out_ref[...] = in_ref[0, :, _IDX]


def kernel(input_):
    x = input_.reshape(_N // _ROWS, _ROWS, _D)
    out = pl.pallas_call(
        _tc_body,
        grid=(_N // _ROWS,),
        in_specs=[pl.BlockSpec((1, _ROWS, 128), lambda i: (i, 0, 0))],
        out_specs=pl.BlockSpec((_ROWS,), lambda i: (i,)),
        out_shape=jax.ShapeDtypeStruct((_N,), jnp.float32),
    )(x)
    return out.reshape(_B, _S, 1)


# speedup vs baseline: 1.1198x; 1.1198x over previous
import jax
import jax.numpy as jnp
from jax.experimental import pallas as pl

_B, _S, _D = 4, 4096, 2048
_N = _B * _S
_IDX = 3
_ROWS = 512


def _tc_body(in_ref, out_ref):
    strip = in_ref[0, :, 0:8]
    out_ref[...] = jnp.swapaxes(strip, 0, 1)[_IDX]


def kernel(input_):
    x = input_.reshape(_N // _ROWS, _ROWS, _D)
    out = pl.pallas_call(
        _tc_body,
        grid=(_N // _ROWS,),
        in_specs=[pl.BlockSpec((1, _ROWS, 128), lambda i: (i, 0, 0))],
        out_specs=pl.BlockSpec((_ROWS,), lambda i: (i,)),
        out_shape=jax.ShapeDtypeStruct((_N,), jnp.float32),
    )(x)
    return out.reshape(_B, _S, 1)


# speedup vs baseline: 2.7933x; 2.4944x over previous
import jax
import jax.numpy as jnp
from jax.experimental import pallas as pl

_B, _S, _D = 4, 4096, 2048
_N = _B * _S
_IDX = 3
_ROWS = 2048


def _tc_body(in_ref, out_ref):
    strip = in_ref[0, :, 0:8]
    out_ref[...] = jnp.swapaxes(strip, 0, 1)[_IDX]


def kernel(input_):
    x = input_.reshape(_N // _ROWS, _ROWS, _D)
    out = pl.pallas_call(
        _tc_body,
        grid=(_N // _ROWS,),
        in_specs=[pl.BlockSpec((1, _ROWS, 128), lambda i: (i, 0, 0))],
        out_specs=pl.BlockSpec((_ROWS,), lambda i: (i,)),
        out_shape=jax.ShapeDtypeStruct((_N,), jnp.float32),
    )(x)
    return out.reshape(_B, _S, 1)


# speedup vs baseline: 3.7032x; 1.3257x over previous
import jax
import jax.numpy as jnp
from jax.experimental import pallas as pl

_B, _S, _D = 4, 4096, 2048
_N = _B * _S
_IDX = 3
_ROWS = 4096


def _tc_body(in_ref, out_ref):
    strip = in_ref[0, :, 0:8]
    out_ref[...] = jnp.swapaxes(strip, 0, 1)[_IDX]


def kernel(input_):
    x = input_.reshape(_N // _ROWS, _ROWS, _D)
    out = pl.pallas_call(
        _tc_body,
        grid=(_N // _ROWS,),
        in_specs=[pl.BlockSpec((1, _ROWS, 128), lambda i: (i, 0, 0))],
        out_specs=pl.BlockSpec((_ROWS,), lambda i: (i,)),
        out_shape=jax.ShapeDtypeStruct((_N,), jnp.float32),
    )(x)
    return out.reshape(_B, _S, 1)


# speedup vs baseline: 4.0253x; 1.0870x over previous
import jax
import jax.numpy as jnp
from jax.experimental import pallas as pl

_B, _S, _D = 4, 4096, 2048
_N = _B * _S
_IDX = 3
_ROWS = 8192


def _tc_body(in_ref, out_ref):
    strip = in_ref[0, :, 0:8]
    out_ref[...] = jnp.swapaxes(strip, 0, 1)[_IDX]


def kernel(input_):
    x = input_.reshape(_N // _ROWS, _ROWS, _D)
    out = pl.pallas_call(
        _tc_body,
        grid=(_N // _ROWS,),
        in_specs=[pl.BlockSpec((1, _ROWS, 128), lambda i: (i, 0, 0))],
        out_specs=pl.BlockSpec((_ROWS,), lambda i: (i,)),
        out_shape=jax.ShapeDtypeStruct((_N,), jnp.float32),
    )(x)
    return out.reshape(_B, _S, 1)
